# Initial kernel scaffold; baseline (speedup 1.0000x reference)
import jax
import jax.numpy as jnp
from jax.experimental import pallas as pl


def kernel(input_embeds, mask, query, Wq_w, Wq_b, Wk_w, Wk_b, Wout, ln_w, ln_b):
    raise NotImplementedError("write your pallas kernel here")


# speedup vs baseline: 1.6882x; 1.6882x over previous
import math
import functools

import jax
import jax.numpy as jnp
from jax.experimental import pallas as pl
from jax.experimental.pallas import tpu as pltpu


def _attn_pool_kernel(x_ref, maskf_ref, query_ref, Wq_w_ref, Wq_b_ref,
                      Wk_w_ref, Wk_b_ref, Wout_ref, ln_w_ref, ln_b_ref,
                      out_ref, g_all_ref, qv_ref, c_ref):
    b = pl.program_id(0)
    nb = pl.num_programs(0)
    H, D = query_ref.shape
    P = Wq_w_ref.shape[0]
    B = out_ref.shape[0]

    @pl.when(b == 0)
    def _prep():
        qq = jax.lax.dot_general(
            query_ref[...], Wq_w_ref[...], (((1,), (1,)), ((), ())),
            preferred_element_type=jnp.float32) + Wq_b_ref[...]
        qv_ref[...] = jax.lax.dot_general(
            qq, Wk_w_ref[...], (((1,), (0,)), ((), ())),
            preferred_element_type=jnp.float32)
        c_ref[...] = jnp.sum(qq * Wk_b_ref[...], axis=1, keepdims=True)

    x = x_ref[0]
    inv_sqrt_p = 1.0 / math.sqrt(P)
    score = (jax.lax.dot_general(
        qv_ref[...], x, (((1,), (1,)), ((), ())),
        preferred_element_type=jnp.float32) + c_ref[...]) * inv_sqrt_p

    maskf = maskf_ref[0]
    neg = jnp.finfo(jnp.float32).min
    score = jnp.where(maskf > 0.0, score, neg)

    m = jnp.max(score, axis=1, keepdims=True)
    e = jnp.exp(score - m)
    s1 = jnp.sum(e, axis=1, keepdims=True)
    prob = e / s1
    prob = prob * maskf
    s2 = jnp.sum(prob, axis=1, keepdims=True) + 1e-6
    prob = prob / s2

    g_all_ref[b] = jax.lax.dot_general(
        prob, x, (((1,), (0,)), ((), ())),
        preferred_element_type=jnp.float32)

    @pl.when(b == nb - 1)
    def _finalize():
        acc = jnp.zeros((B, D), jnp.float32)
        for h in range(H):
            acc = acc + jax.lax.dot_general(
                g_all_ref[:, h, :], Wout_ref[:, h * D:(h + 1) * D],
                (((1,), (1,)), ((), ())), preferred_element_type=jnp.float32)
        mu = jnp.mean(acc, axis=1, keepdims=True)
        var = jnp.mean((acc - mu) ** 2, axis=1, keepdims=True)
        out_ref[...] = ((acc - mu) * jax.lax.rsqrt(var + 1e-5)
                        * ln_w_ref[...] + ln_b_ref[...])


@functools.partial(jax.jit, static_argnames=())
def kernel(input_embeds, mask, query, Wq_w, Wq_b, Wk_w, Wk_b, Wout, ln_w, ln_b):
    B, S, D = input_embeds.shape
    H = query.shape[0]
    P = Wq_w.shape[0]

    maskf = mask.astype(jnp.float32).reshape(B, 1, S)

    grid = (B,)
    out = pl.pallas_call(
        _attn_pool_kernel,
        grid=grid,
        in_specs=[
            pl.BlockSpec((1, S, D), lambda b: (b, 0, 0)),
            pl.BlockSpec((1, 1, S), lambda b: (b, 0, 0)),
            pl.BlockSpec((H, D), lambda b: (0, 0)),
            pl.BlockSpec((P, D), lambda b: (0, 0)),
            pl.BlockSpec((1, P), lambda b: (0, 0)),
            pl.BlockSpec((P, D), lambda b: (0, 0)),
            pl.BlockSpec((1, P), lambda b: (0, 0)),
            pl.BlockSpec((D, H * D), lambda b: (0, 0)),
            pl.BlockSpec((1, D), lambda b: (0, 0)),
            pl.BlockSpec((1, D), lambda b: (0, 0)),
        ],
        out_specs=pl.BlockSpec((B, D), lambda b: (0, 0)),
        out_shape=jax.ShapeDtypeStruct((B, D), jnp.float32),
        scratch_shapes=[
            pltpu.VMEM((B, H, D), jnp.float32),
            pltpu.VMEM((H, D), jnp.float32),
            pltpu.VMEM((H, 1), jnp.float32),
        ],
        compiler_params=pltpu.CompilerParams(
            dimension_semantics=("arbitrary",),
        ),
    )(input_embeds, maskf, query, Wq_w, Wq_b.reshape(1, P), Wk_w,
      Wk_b.reshape(1, P), Wout, ln_w.reshape(1, D), ln_b.reshape(1, D))
    return out
